# baseline (speedup 1.0000x reference)
import functools

import jax
import jax.numpy as jnp
from jax import lax
from jax.experimental import pallas as pl
from jax.experimental.pallas import tpu as pltpu
from jax.experimental.pallas import tpu_sc as plsc

NF = 26
V = 100000
E = 32
B = 4096

NC = 2
NS = 16
L = 16
NW = NC * NS

B_PER_W = B // NW
ROWS_PER_W = B_PER_W * NF
NB = 4
CHUNK_ROWS = NB * NF
NCHUNKS = B_PER_W // NB
BE = E * NF
VECS_PER_CHUNK = NB * BE // L
VECS_PER_ELEM = BE // L


def _body(ids_hbm, tab_hbm, out_hbm, idx_v, g_v, o_v, pat_v, gsem):
    wid = lax.axis_index("s") * NC + lax.axis_index("c")
    row_base = wid * ROWS_PER_W

    pltpu.sync_copy(ids_hbm.at[pl.ds(row_base, ROWS_PER_W)], idx_v)

    def pat_step(m, _):
        f = m // 2
        e0 = (m % 2) * L
        lanes = lax.iota(jnp.int32, L)
        pat_v[pl.ds(m * L, L)] = (e0 + lanes) * NF + f
        return 0

    lax.fori_loop(0, VECS_PER_ELEM, pat_step, 0, unroll=False)

    def idx_step(i, _):
        pos = i * L + lax.iota(jnp.int32, L)
        f = lax.rem(pos, NF)
        idx_v[pl.ds(i * L, L)] = idx_v[pl.ds(i * L, L)] + f * V
        return 0

    lax.fori_loop(0, ROWS_PER_W // L, idx_step, 0, unroll=False)

    def chunk_step(c, _):
        pltpu.async_copy(
            tab_hbm.at[idx_v.at[pl.ds(c * CHUNK_ROWS, CHUNK_ROWS)]],
            g_v,
            gsem,
        ).wait()

        def t_step(t, _):
            r = t // 2
            e0 = (t % 2) * L
            j = t // VECS_PER_ELEM
            m = t % VECS_PER_ELEM
            vals = g_v[r, pl.ds(e0, L)]
            dst = pat_v[pl.ds(m * L, L)] + j * BE
            plsc.store_scatter(o_v, [dst], vals)
            return 0

        lax.fori_loop(0, VECS_PER_CHUNK, t_step, 0, unroll=False)

        out_off = (wid * B_PER_W + c * NB) * BE
        pltpu.sync_copy(o_v, out_hbm.at[pl.ds(out_off, NB * BE)])
        return 0

    lax.fori_loop(0, NCHUNKS, chunk_step, 0, unroll=False)


@jax.jit
def kernel(ids, tables):
    ids_flat = ids.reshape(B * NF)
    tab_flat = tables.reshape(NF * V, E)

    mesh = plsc.VectorSubcoreMesh(
        core_axis_name="c", subcore_axis_name="s", num_cores=NC, num_subcores=NS
    )
    out_flat = pl.kernel(
        _body,
        out_type=jax.ShapeDtypeStruct((B * BE,), jnp.float32),
        mesh=mesh,
        compiler_params=pltpu.CompilerParams(
            needs_layout_passes=False, use_tc_tiling_on_sc=False
        ),
        scratch_types=[
            pltpu.VMEM((ROWS_PER_W,), jnp.int32),
            pltpu.VMEM((CHUNK_ROWS, E), jnp.float32),
            pltpu.VMEM((NB * BE,), jnp.float32),
            pltpu.VMEM((BE,), jnp.int32),
            pltpu.SemaphoreType.DMA,
        ],
    )(ids_flat, tab_flat)
    return out_flat.reshape(B, E, NF)
